# baseline (speedup 1.0000x reference)
import functools

import jax
import jax.numpy as jnp
from jax import lax
from jax.experimental import pallas as pl
from jax.experimental.pallas import tpu as pltpu
from jax.experimental.pallas import tpu_sc as plsc

_V = 1000
_NTOT = 1024 * 50
_NW = 32
_PER_W = _NTOT // _NW
_CH = 16
_NCH = _PER_W // _CH
_LAG = 16
_ROWS_PER_S = 62


def _sc_gather(table, idx_flat):
  mesh = plsc.VectorSubcoreMesh(core_axis_name="c", subcore_axis_name="s")

  @functools.partial(
      pl.kernel,
      mesh=mesh,
      out_type=jax.ShapeDtypeStruct((_NTOT, _V), jnp.float32),
      compiler_params=pltpu.CompilerParams(use_tc_tiling_on_sc=False),
      scratch_types=[
          pltpu.VMEM_SHARED((_V, _V), jnp.float32),
          pltpu.VMEM((_PER_W,), jnp.int32),
          pltpu.SemaphoreType.DMA,
      ],
  )
  def k(table_hbm, idx_hbm, out_hbm, table_sp, idx_v, sem):
    sid = lax.axis_index("s")
    wid = sid * 2 + lax.axis_index("c")
    base = wid * _PER_W

    r0 = sid * _ROWS_PER_S
    pltpu.sync_copy(table_hbm.at[pl.ds(r0, _ROWS_PER_S)],
                    table_sp.at[pl.ds(r0, _ROWS_PER_S)])

    @pl.when(sid == 0)
    def _tail():
      pltpu.sync_copy(table_hbm.at[pl.ds(_ROWS_PER_S * 16, 8)],
                      table_sp.at[pl.ds(_ROWS_PER_S * 16, 8)])

    pltpu.sync_copy(idx_hbm.at[pl.ds(base, _PER_W)], idx_v)
    plsc.subcore_barrier()

    def issue_chunk(c):
      chunk = idx_v[pl.ds(c * _CH, _CH)]
      for l in range(_CH):
        row = chunk[l]
        pltpu.async_copy(table_sp.at[row], out_hbm.at[base + c * _CH + l],
                         sem)

    def wait_one():
      pltpu.make_async_copy(table_sp.at[0], out_hbm.at[0], sem).wait()

    def prime(c, carry):
      issue_chunk(c)
      return carry

    lax.fori_loop(0, _LAG, prime, 0)

    def body(c, carry):
      for _ in range(_CH):
        wait_one()
      issue_chunk(c)
      return carry

    lax.fori_loop(_LAG, _NCH, body, 0)

    def drain(i, carry):
      wait_one()
      return carry

    lax.fori_loop(0, _LAG * _CH, drain, 0)

  return k(table, idx_flat)


def kernel(x, prob):
  idx = x.reshape(-1).astype(jnp.int32)
  out = _sc_gather(prob, idx)
  return out.reshape(x.shape[0], x.shape[1], _V)
